# baseline (speedup 1.0000x reference)
import functools

import jax
import jax.numpy as jnp
from jax import lax
from jax.experimental import pallas as pl
from jax.experimental.pallas import tpu as pltpu
from jax.experimental.pallas import tpu_sc as plsc

_LANES = 16
_POS_PER_CHUNK = 64


def kernel(x, pe_table):
    B, S, D = x.shape
    ML = pe_table.shape[0]
    x2 = x.reshape(B, S * D)
    pe2 = pe_table.reshape(ML * D)

    info = plsc.get_sparse_core_info()
    NC, NS = info.num_cores, info.num_subcores
    NW = NC * NS
    pos_per_w = S // NW
    P = _POS_PER_CHUNK
    CH = P * D
    n_chunks = pos_per_w // P

    @functools.partial(
        pl.kernel,
        mesh=plsc.VectorSubcoreMesh(core_axis_name="c", subcore_axis_name="s"),
        out_type=jax.ShapeDtypeStruct((B, S * D), jnp.float32),
        scratch_types=[
            pltpu.VMEM((CH,), jnp.float32),
            pltpu.VMEM((CH,), jnp.float32),
        ],
    )
    def sc_add(x_hbm, pe_hbm, out_hbm, pebuf, xbuf):
        wid = lax.axis_index("s") * NC + lax.axis_index("c")
        base = wid * pos_per_w * D

        def chunk_body(ci, carry):
            off = base + ci * CH
            pltpu.sync_copy(pe_hbm.at[pl.ds(off, CH)], pebuf)
            for b in range(B):
                pltpu.sync_copy(x_hbm.at[b, pl.ds(off, CH)], xbuf)

                def add_body(i, c2):
                    sl = pl.ds(i * _LANES, _LANES)
                    xbuf[sl] = xbuf[sl] + pebuf[sl]
                    return c2

                lax.fori_loop(0, CH // _LANES, add_body, 0)
                pltpu.sync_copy(xbuf, out_hbm.at[b, pl.ds(off, CH)])
            return carry

        lax.fori_loop(0, n_chunks, chunk_body, 0)

    out = sc_add(x2, pe2)
    return out.reshape(B, S, D)


# speedup vs baseline: 1.5150x; 1.5150x over previous
import functools

import jax
import jax.numpy as jnp
from jax import lax
from jax.experimental import pallas as pl
from jax.experimental.pallas import tpu as pltpu
from jax.experimental.pallas import tpu_sc as plsc

_LANES = 16
_POS_PER_CHUNK = 64


def kernel(x, pe_table):
    B, S, D = x.shape
    ML = pe_table.shape[0]
    x2 = x.reshape(B, S * D)
    pe2 = pe_table.reshape(ML * D)

    info = plsc.get_sparse_core_info()
    NC, NS = info.num_cores, info.num_subcores
    NW = NC * NS
    pos_per_w = S // NW
    P = _POS_PER_CHUNK
    CH = P * D
    n_chunks = pos_per_w // P

    @functools.partial(
        pl.kernel,
        mesh=plsc.VectorSubcoreMesh(core_axis_name="c", subcore_axis_name="s"),
        out_type=jax.ShapeDtypeStruct((B, S * D), jnp.float32),
        scratch_types=[
            pltpu.VMEM((CH,), jnp.float32),
            pltpu.VMEM((CH,), jnp.float32),
        ],
    )
    def sc_add(x_hbm, pe_hbm, out_hbm, pebuf, xbuf):
        wid = lax.axis_index("s") * NC + lax.axis_index("c")
        base = wid * pos_per_w * D

        def chunk_body(ci, carry):
            off = base + ci * CH
            pltpu.sync_copy(pe_hbm.at[pl.ds(off, CH)], pebuf)
            for b in range(B):
                pltpu.sync_copy(x_hbm.at[b, pl.ds(off, CH)], xbuf)

                @plsc.parallel_loop(0, CH, step=_LANES, unroll=8)
                def add_body(i):
                    sl = pl.ds(i, _LANES)
                    xbuf[sl] = xbuf[sl] + pebuf[sl]

                pltpu.sync_copy(xbuf, out_hbm.at[b, pl.ds(off, CH)])
            return carry

        lax.fori_loop(0, n_chunks, chunk_body, 0)

    out = sc_add(x2, pe2)
    return out.reshape(B, S, D)


# speedup vs baseline: 1.7934x; 1.1838x over previous
import functools

import jax
import jax.numpy as jnp
from jax import lax
from jax.experimental import pallas as pl
from jax.experimental.pallas import tpu as pltpu
from jax.experimental.pallas import tpu_sc as plsc

_LANES = 16
_POS_PER_CHUNK = 32
_UNROLL = 8


def kernel(x, pe_table):
    B, S, D = x.shape
    ML = pe_table.shape[0]
    x2 = x.reshape(B, S * D)
    pe2 = pe_table.reshape(ML * D)

    info = plsc.get_sparse_core_info()
    NC, NS = info.num_cores, info.num_subcores
    NW = NC * NS
    pos_per_w = S // NW
    P = _POS_PER_CHUNK
    CH = P * D
    n_chunks = pos_per_w // P
    n_steps = n_chunks * B

    @functools.partial(
        pl.kernel,
        mesh=plsc.VectorSubcoreMesh(core_axis_name="c", subcore_axis_name="s"),
        out_type=jax.ShapeDtypeStruct((B, S * D), jnp.float32),
        scratch_types=[
            pltpu.VMEM((CH,), jnp.float32),
            pltpu.VMEM((CH,), jnp.float32),
            pltpu.VMEM((CH,), jnp.float32),
            pltpu.VMEM((CH,), jnp.float32),
            pltpu.SemaphoreType.DMA,
            pltpu.SemaphoreType.DMA,
            pltpu.SemaphoreType.DMA,
            pltpu.SemaphoreType.DMA,
            pltpu.SemaphoreType.DMA,
            pltpu.SemaphoreType.DMA,
        ],
    )
    def sc_add(x_hbm, pe_hbm, out_hbm, xa, xb, pea, peb,
               sem_xa, sem_xb, sem_pea, sem_peb, sem_oa, sem_ob):
        wid = lax.axis_index("s") * NC + lax.axis_index("c")
        base = wid * pos_per_w * D

        xbufs = (xa, xb)
        pebufs = (pea, peb)
        xsems = (sem_xa, sem_xb)
        pesems = (sem_pea, sem_peb)
        osems = (sem_oa, sem_ob)

        handles = {}

        def off(ci):
            return base + ci * CH

        handles[("x", 0)] = pltpu.async_copy(
            x_hbm.at[0, pl.ds(off(0), CH)], xbufs[0], xsems[0])
        handles[("pe", 0)] = pltpu.async_copy(
            pe_hbm.at[pl.ds(off(0), CH)], pebufs[0], pesems[0])

        for k in range(n_steps):
            ci, b = divmod(k, B)
            xi = k % 2
            pi = ci % 2

            if k + 1 < n_steps:
                ni = (k + 1) % 2
                if ("o", k - 1) in handles:
                    handles[("o", k - 1)].wait()
                ci2, b2 = divmod(k + 1, B)
                handles[("x", k + 1)] = pltpu.async_copy(
                    x_hbm.at[b2, pl.ds(off(ci2), CH)], xbufs[ni], xsems[ni])

            handles[("x", k)].wait()
            if b == 0:
                handles[("pe", ci)].wait()

            xbuf = xbufs[xi]
            pebuf = pebufs[pi]

            @plsc.parallel_loop(0, CH, step=_LANES, unroll=_UNROLL)
            def add_body(i):
                sl = pl.ds(i, _LANES)
                xbuf[sl] = xbuf[sl] + pebuf[sl]

            if b == 0 and ci + 1 < n_chunks:
                npi = (ci + 1) % 2
                handles[("pe", ci + 1)] = pltpu.async_copy(
                    pe_hbm.at[pl.ds(off(ci + 1), CH)], pebufs[npi], pesems[npi])

            handles[("o", k)] = pltpu.async_copy(
                xbuf, out_hbm.at[b, pl.ds(off(ci), CH)], osems[xi])

        handles[("o", n_steps - 2)].wait()
        handles[("o", n_steps - 1)].wait()

    out = sc_add(x2, pe2)
    return out.reshape(B, S, D)


# speedup vs baseline: 4.0877x; 2.2793x over previous
import functools

import jax
import jax.numpy as jnp
from jax import lax
from jax.experimental import pallas as pl
from jax.experimental.pallas import tpu as pltpu
from jax.experimental.pallas import tpu_sc as plsc

_LANES = 16
_POS_PER_CHUNK = 32


def kernel(x, pe_table):
    B, S, D = x.shape
    x2 = x.reshape(B * S, D)

    info = plsc.get_sparse_core_info()
    NC, NS = info.num_cores, info.num_subcores
    NW = NC * NS
    pos_per_w = S // NW
    P = _POS_PER_CHUNK
    n_chunks = pos_per_w // P
    n_steps = n_chunks * B

    @functools.partial(
        pl.kernel,
        mesh=plsc.VectorSubcoreMesh(core_axis_name="c", subcore_axis_name="s"),
        out_type=jax.ShapeDtypeStruct((B * S, D), jnp.float32),
        scratch_types=[
            pltpu.VMEM((P, D), jnp.float32),
            pltpu.VMEM((P, D), jnp.float32),
            pltpu.VMEM((P, D), jnp.float32),
            pltpu.VMEM((P, D), jnp.float32),
            pltpu.SemaphoreType.DMA,
            pltpu.SemaphoreType.DMA,
            pltpu.SemaphoreType.DMA,
            pltpu.SemaphoreType.DMA,
            pltpu.SemaphoreType.DMA,
            pltpu.SemaphoreType.DMA,
        ],
    )
    def sc_add(x_hbm, pe_hbm, out_hbm, xa, xb, pea, peb,
               sem_xa, sem_xb, sem_pea, sem_peb, sem_oa, sem_ob):
        wid = lax.axis_index("s") * NC + lax.axis_index("c")
        base_pos = wid * pos_per_w

        xbufs = (xa, xb)
        pebufs = (pea, peb)
        xsems = (sem_xa, sem_xb)
        pesems = (sem_pea, sem_peb)
        osems = (sem_oa, sem_ob)

        handles = {}

        def pos0(ci):
            return base_pos + ci * P

        handles[("x", 0)] = pltpu.async_copy(
            x_hbm.at[pl.ds(pos0(0), P), :], xbufs[0], xsems[0])
        handles[("pe", 0)] = pltpu.async_copy(
            pe_hbm.at[pl.ds(pos0(0), P), :], pebufs[0], pesems[0])

        for k in range(n_steps):
            ci, b = divmod(k, B)
            xi = k % 2
            pi = ci % 2

            if k + 1 < n_steps:
                ni = (k + 1) % 2
                if ("o", k - 1) in handles:
                    handles[("o", k - 1)].wait()
                ci2, b2 = divmod(k + 1, B)
                handles[("x", k + 1)] = pltpu.async_copy(
                    x_hbm.at[pl.ds(b2 * S + pos0(ci2), P), :],
                    xbufs[ni], xsems[ni])

            handles[("x", k)].wait()
            if b == 0:
                handles[("pe", ci)].wait()

            xbuf = xbufs[xi]
            pebuf = pebufs[pi]

            def row_body(r, carry):
                @plsc.parallel_loop(0, D, step=_LANES, unroll=8)
                def slice_body(c):
                    sl = pl.ds(c, _LANES)
                    xbuf[r, sl] = xbuf[r, sl] + pebuf[r, sl]

                return carry

            lax.fori_loop(0, P, row_body, 0)

            if b == 0 and ci + 1 < n_chunks:
                npi = (ci + 1) % 2
                handles[("pe", ci + 1)] = pltpu.async_copy(
                    pe_hbm.at[pl.ds(pos0(ci + 1), P), :],
                    pebufs[npi], pesems[npi])

            handles[("o", k)] = pltpu.async_copy(
                xbuf, out_hbm.at[pl.ds(b * S + pos0(ci), P), :], osems[xi])

        handles[("o", n_steps - 2)].wait()
        handles[("o", n_steps - 1)].wait()

    out = sc_add(x2, pe_table)
    return out.reshape(B, S, D)
